# Initial kernel scaffold; baseline (speedup 1.0000x reference)
import jax
import jax.numpy as jnp
from jax.experimental import pallas as pl


def kernel(q_xyz, q_feat, kv_xyz, kv_feat, Wqkv, bqkv, Wo, bo, W1, b1, W2, b2, g1, be1, g2, be2):
    raise NotImplementedError("write your pallas kernel here")


# speedup vs baseline: 12.7462x; 12.7462x over previous
import functools

import jax
import jax.numpy as jnp
import numpy as np
from jax.experimental import pallas as pl
from jax.experimental.pallas import tpu as pltpu
from jax.experimental.pallas import tpu_sc as plsc

RADIUS = 0.1
K = 8
H = 8


TA = 256


def _topk_body(qpad_ref, kpadT_ref, idx_ref, vmask_ref):
    qp = qpad_ref[0]
    kp = kpadT_ref[0]
    m = kp.shape[-1]
    dt = jax.lax.dot_general(qp, kp, (((1,), (0,)), ((), ())),
                             preferred_element_type=jnp.float32)
    qn = jnp.sum(qp * qp, axis=1, keepdims=True)
    kn = jnp.sum(kp * kp, axis=0, keepdims=True)
    d2 = (qn + kn) - 2.0 * dt
    dist = jnp.sqrt(jnp.maximum(d2, 1e-12))
    dist = jnp.where(dist <= RADIUS, dist, jnp.inf)
    iota = jax.lax.broadcasted_iota(jnp.int32, dist.shape, 1)
    idxs = []
    valids = []
    for _ in range(K):
        mn = jnp.min(dist, axis=1, keepdims=True)
        cand = jnp.where(dist == mn, iota, m)
        ij = jnp.min(cand, axis=1)
        valids.append(jnp.where(mn[:, 0] < jnp.inf, 1.0, 0.0))
        idxs.append(ij)
        dist = jnp.where(iota == ij[:, None], jnp.inf, dist)
    idx_ref[0] = jnp.stack(idxs, axis=0)
    vmask_ref[0] = jnp.stack(valids, axis=0)


def _topk_call(qaug, kaugT):
    B, N, _ = qaug.shape
    M = kaugT.shape[-1]
    grid = (B, N // TA)
    return pl.pallas_call(
        _topk_body,
        grid=grid,
        in_specs=[
            pl.BlockSpec((1, TA, 8), lambda b, t: (b, t, 0)),
            pl.BlockSpec((1, 8, M), lambda b, t: (b, 0, 0)),
        ],
        out_specs=[
            pl.BlockSpec((1, K, TA), lambda b, t: (b, 0, t)),
            pl.BlockSpec((1, K, TA), lambda b, t: (b, 0, t)),
        ],
        out_shape=[
            jax.ShapeDtypeStruct((B, K, N), jnp.int32),
            jax.ShapeDtypeStruct((B, K, N), jnp.float32),
        ],
    )(qaug, kaugT)


_SC_CORES = 2
_SC_SUBCORES = 16


def _sc_gather_call(table, idx2d):
    n_idx_rows, _ = idx2d.shape
    C = table.shape[-1]
    nw = _SC_CORES * _SC_SUBCORES
    total_rows = n_idx_rows * 128
    rows_per_w = total_rows // nw
    idx_rows_per_w = rows_per_w // 128
    n_chunks = 2
    fires = idx_rows_per_w // n_chunks
    rows_per_chunk = rows_per_w // n_chunks

    mesh = plsc.VectorSubcoreMesh(core_axis_name="c", subcore_axis_name="s")

    @functools.partial(
        pl.kernel,
        mesh=mesh,
        compiler_params=pltpu.CompilerParams(use_tc_tiling_on_sc=False),
        out_type=jax.ShapeDtypeStruct((total_rows, C), jnp.float32),
        scratch_types=[
            pltpu.VMEM((idx_rows_per_w, 128), jnp.int32),
            pltpu.VMEM((rows_per_chunk, C), jnp.float32),
            pltpu.SemaphoreType.DMA,
        ],
    )
    def gather_kernel(table_hbm, idx_hbm, out_hbm, idx_v, rows_v, sem):
        wid = jax.lax.axis_index("s") * _SC_CORES + jax.lax.axis_index("c")
        pltpu.sync_copy(idx_hbm.at[pl.ds(wid * idx_rows_per_w, idx_rows_per_w)],
                        idx_v)
        base_out = wid * rows_per_w
        for c in range(n_chunks):
            copies = [
                pltpu.async_copy(
                    table_hbm.at[idx_v.at[c * fires + i]],
                    rows_v.at[pl.ds(i * 128, 128)],
                    sem,
                )
                for i in range(fires)
            ]
            for cp in copies:
                cp.wait()
            pltpu.sync_copy(
                rows_v,
                out_hbm.at[pl.ds(base_out + c * rows_per_chunk, rows_per_chunk)])

    return gather_kernel(table, idx2d)


TB = 256


def _fuse_body(q_ref, nb_ref, vm_ref, wqkvT_ref, bqkv_ref, woT_ref, bo_ref,
               w1T_ref, b1_ref, w2T_ref, b2_ref, g1_ref, be1_ref, g2_ref,
               be2_ref, out_ref):
    T = q_ref.shape[1]
    C = q_ref.shape[2]
    dh = C // H
    x0 = q_ref[0]
    wqkvT = wqkvT_ref[...]
    bqkv = bqkv_ref[...]

    qkv_s = jax.lax.dot_general(x0, wqkvT, (((1,), (0,)), ((), ())),
                                preferred_element_type=jnp.float32) + bqkv
    q0 = qkv_s[:, :C]
    k_s = qkv_s[:, C:2 * C]
    v_s = qkv_s[:, 2 * C:]

    nb = nb_ref[0]
    nb2 = nb.reshape(K * T, C)
    kvn = jax.lax.dot_general(nb2, wqkvT[:, C:], (((1,), (0,)), ((), ())),
                              preferred_element_type=jnp.float32) + bqkv[:, C:]
    k_n = kvn[:, :C].reshape(K, T, C)
    v_n = kvn[:, C:].reshape(K, T, C)

    k_all = jnp.concatenate([k_s[None], k_n], axis=0)
    v_all = jnp.concatenate([v_s[None], v_n], axis=0)

    gi = jax.lax.broadcasted_iota(jnp.int32, (C, H), 0) // dh
    gh = jax.lax.broadcasted_iota(jnp.int32, (C, H), 1)
    G = (gi == gh).astype(jnp.float32)
    gti = jax.lax.broadcasted_iota(jnp.int32, (H, C), 1) // dh
    gth = jax.lax.broadcasted_iota(jnp.int32, (H, C), 0)
    Gt = (gti == gth).astype(jnp.float32)

    prod = q0[None] * k_all
    s = jax.lax.dot_general(prod.reshape((K + 1) * T, C), G,
                            (((1,), (0,)), ((), ())),
                            preferred_element_type=jnp.float32)
    s = s.reshape(K + 1, T, H) * jnp.float32(1.0 / np.sqrt(dh))

    vm = vm_ref[0]
    mask9 = jnp.concatenate([jnp.ones((1, T), jnp.float32), vm], axis=0)
    s = jnp.where(mask9[:, :, None] > 0.0, s, jnp.float32(-1e30))
    mx = jnp.max(s, axis=0, keepdims=True)
    e = jnp.exp(s - mx)
    w = e / jnp.sum(e, axis=0, keepdims=True)

    wb = jax.lax.dot_general(w.reshape((K + 1) * T, H), Gt,
                             (((1,), (0,)), ((), ())),
                             preferred_element_type=jnp.float32)
    o = jnp.sum(wb.reshape(K + 1, T, C) * v_all, axis=0)

    attn = jax.lax.dot_general(o, woT_ref[...], (((1,), (0,)), ((), ())),
                               preferred_element_type=jnp.float32) + bo_ref[...]

    x = x0 + attn
    mu = jnp.mean(x, axis=-1, keepdims=True)
    xc = x - mu
    var = jnp.mean(xc * xc, axis=-1, keepdims=True)
    xn = xc / jnp.sqrt(var + 1e-5) * g1_ref[...] + be1_ref[...]

    h1 = jax.lax.dot_general(xn, w1T_ref[...], (((1,), (0,)), ((), ())),
                             preferred_element_type=jnp.float32) + b1_ref[...]
    h1 = jnp.maximum(h1, 0.0)
    f = jax.lax.dot_general(h1, w2T_ref[...], (((1,), (0,)), ((), ())),
                            preferred_element_type=jnp.float32) + b2_ref[...]

    x2 = xn + f
    mu2 = jnp.mean(x2, axis=-1, keepdims=True)
    xc2 = x2 - mu2
    var2 = jnp.mean(xc2 * xc2, axis=-1, keepdims=True)
    y = xc2 / jnp.sqrt(var2 + 1e-5) * g2_ref[...] + be2_ref[...]

    out_ref[0] = y + x0


def _fuse_call(q_feat, neigh, vmask, wqkvT, bqkv, woT, bo, w1T, b1, w2T, b2,
               g1, be1, g2, be2):
    B, N, C = q_feat.shape
    FF = w1T.shape[-1]
    grid = (B, N // TB)
    full = lambda shape: pl.BlockSpec(shape, lambda b, t: tuple(0 for _ in shape))
    return pl.pallas_call(
        _fuse_body,
        grid=grid,
        in_specs=[
            pl.BlockSpec((1, TB, C), lambda b, t: (b, t, 0)),
            pl.BlockSpec((1, K, TB, C), lambda b, t: (b, 0, t, 0)),
            pl.BlockSpec((1, K, TB), lambda b, t: (b, 0, t)),
            full((C, 3 * C)),
            full((1, 3 * C)),
            full((C, C)),
            full((1, C)),
            full((C, FF)),
            full((1, FF)),
            full((FF, C)),
            full((1, C)),
            full((1, C)),
            full((1, C)),
            full((1, C)),
            full((1, C)),
        ],
        out_specs=pl.BlockSpec((1, TB, C), lambda b, t: (b, t, 0)),
        out_shape=jax.ShapeDtypeStruct((B, N, C), jnp.float32),
    )(q_feat, neigh, vmask, wqkvT, bqkv, woT, bo, w1T, b1, w2T, b2,
      g1, be1, g2, be2)


def kernel(q_xyz, q_feat, kv_xyz, kv_feat, Wqkv, bqkv, Wo, bo, W1, b1, W2, b2,
           g1, be1, g2, be2):
    B, N, C = q_feat.shape
    M = kv_xyz.shape[1]

    qpad = jnp.concatenate([q_xyz, jnp.zeros((B, N, 5), jnp.float32)], axis=-1)
    kpad = jnp.concatenate([kv_xyz, jnp.zeros((B, M, 5), jnp.float32)], axis=-1)
    kpadT = jnp.transpose(kpad, (0, 2, 1))

    idxT, vmaskT = _topk_call(qpad, kpadT)

    flat_idx = (idxT + (jnp.arange(B, dtype=jnp.int32) * M)[:, None, None])
    idx2d = flat_idx.reshape(B * K * N // 128, 128)
    table = kv_feat.reshape(B * M, C)

    neigh_flat = _sc_gather_call(table, idx2d)
    neigh = neigh_flat.reshape(B, K, N, C)

    return _fuse_call(
        q_feat, neigh, vmaskT,
        jnp.transpose(Wqkv), bqkv.reshape(1, 3 * C),
        jnp.transpose(Wo), bo.reshape(1, C),
        jnp.transpose(W1), b1.reshape(1, W1.shape[0]),
        jnp.transpose(W2), b2.reshape(1, C),
        g1.reshape(1, C), be1.reshape(1, C),
        g2.reshape(1, C), be2.reshape(1, C),
    )


# speedup vs baseline: 13.5163x; 1.0604x over previous
import functools

import jax
import jax.numpy as jnp
import numpy as np
from jax.experimental import pallas as pl
from jax.experimental.pallas import tpu as pltpu
from jax.experimental.pallas import tpu_sc as plsc

RADIUS = 0.1
K = 8
H = 8


TA = 256


def _topk_body(qpad_ref, kpadT_ref, idx_ref, vmask_ref):
    qp = qpad_ref[0]
    kp = kpadT_ref[0]
    m = kp.shape[-1]
    dt = jax.lax.dot_general(qp, kp, (((1,), (0,)), ((), ())),
                             preferred_element_type=jnp.float32)
    qn = jnp.sum(qp * qp, axis=1, keepdims=True)
    kn = jnp.sum(kp * kp, axis=0, keepdims=True)
    d2 = (qn + kn) - 2.0 * dt
    dist = jnp.sqrt(jnp.maximum(d2, 1e-12))
    dist = jnp.where(dist <= RADIUS, dist, jnp.inf)
    cnt = jnp.sum(jnp.where(dist < jnp.inf, 1.0, 0.0), axis=1)
    iota = jax.lax.broadcasted_iota(jnp.int32, dist.shape, 1)
    idxs = []
    for j in range(K):
        ij = jnp.argmin(dist, axis=1).astype(jnp.int32)
        idxs.append(ij)
        dist = jnp.where(iota == ij[:, None], jnp.inf, dist)
    idx_ref[0] = jnp.stack(idxs, axis=0)
    vmask_ref[0] = jnp.stack(
        [jnp.where(cnt > j, 1.0, 0.0) for j in range(K)], axis=0)


def _topk_call(qaug, kaugT):
    B, N, _ = qaug.shape
    M = kaugT.shape[-1]
    grid = (B, N // TA)
    return pl.pallas_call(
        _topk_body,
        grid=grid,
        in_specs=[
            pl.BlockSpec((1, TA, 8), lambda b, t: (b, t, 0)),
            pl.BlockSpec((1, 8, M), lambda b, t: (b, 0, 0)),
        ],
        out_specs=[
            pl.BlockSpec((1, K, TA), lambda b, t: (b, 0, t)),
            pl.BlockSpec((1, K, TA), lambda b, t: (b, 0, t)),
        ],
        out_shape=[
            jax.ShapeDtypeStruct((B, K, N), jnp.int32),
            jax.ShapeDtypeStruct((B, K, N), jnp.float32),
        ],
    )(qaug, kaugT)


_SC_CORES = 2
_SC_SUBCORES = 16


def _sc_gather_call(table, idx2d):
    n_idx_rows, lanes = idx2d.shape
    C = table.shape[-1]
    nw = _SC_CORES * _SC_SUBCORES
    idx_rows_per_w = n_idx_rows // nw
    desc_per_chunk = 4
    n_chunks = idx_rows_per_w // desc_per_chunk
    chunk_rows = desc_per_chunk * lanes
    rows_per_w = idx_rows_per_w * lanes

    mesh = plsc.VectorSubcoreMesh(core_axis_name="c", subcore_axis_name="s")

    @functools.partial(
        pl.kernel,
        mesh=mesh,
        compiler_params=pltpu.CompilerParams(use_tc_tiling_on_sc=False),
        out_type=jax.ShapeDtypeStruct((n_idx_rows * lanes, C), jnp.float32),
        scratch_types=[
            pltpu.VMEM((idx_rows_per_w, lanes), jnp.int32),
            pltpu.VMEM((chunk_rows, C), jnp.float32),
            pltpu.VMEM((chunk_rows, C), jnp.float32),
            pltpu.SemaphoreType.DMA,
            pltpu.SemaphoreType.DMA,
        ],
    )
    def gather_kernel(table_hbm, idx_hbm, out_hbm, idx_v, rows_a, rows_b, sem_a,
                      sem_b):
        wid = jax.lax.axis_index("s") * _SC_CORES + jax.lax.axis_index("c")
        pltpu.sync_copy(idx_hbm.at[pl.ds(wid * idx_rows_per_w, idx_rows_per_w)],
                        idx_v)
        base_out = wid * rows_per_w
        bufs = [(rows_a, sem_a), (rows_b, sem_b)]

        def fire(c):
            rows_v, sem = bufs[c % 2]
            return [
                pltpu.async_copy(
                    table_hbm.at[idx_v.at[c * desc_per_chunk + i]],
                    rows_v.at[pl.ds(i * lanes, lanes)], sem)
                for i in range(desc_per_chunk)
            ]

        pend = {0: fire(0), 1: fire(1)}
        for c in range(n_chunks):
            for cp in pend.pop(c):
                cp.wait()
            pltpu.sync_copy(bufs[c % 2][0],
                            out_hbm.at[pl.ds(base_out + c * chunk_rows,
                                             chunk_rows)])
            if c + 2 < n_chunks:
                pend[c + 2] = fire(c + 2)

    return gather_kernel(table, idx2d)


TB = 256


def _fuse_body(q_ref, nb_ref, vm_ref, wqkvT_ref, bqkv_ref, woT_ref, bo_ref,
               w1T_ref, b1_ref, w2T_ref, b2_ref, g1_ref, be1_ref, g2_ref,
               be2_ref, out_ref):
    T = q_ref.shape[1]
    C = q_ref.shape[2]
    dh = C // H
    x0 = q_ref[0]
    wqkvT = wqkvT_ref[...]
    bqkv = bqkv_ref[...]

    qkv_s = jax.lax.dot_general(x0, wqkvT, (((1,), (0,)), ((), ())),
                                preferred_element_type=jnp.float32) + bqkv
    q0 = qkv_s[:, :C]
    k_s = qkv_s[:, C:2 * C]
    v_s = qkv_s[:, 2 * C:]

    nb = nb_ref[0]
    nb2 = nb.reshape(K * T, C)
    kvn = jax.lax.dot_general(nb2, wqkvT[:, C:], (((1,), (0,)), ((), ())),
                              preferred_element_type=jnp.float32) + bqkv[:, C:]
    k_n = kvn[:, :C].reshape(K, T, C)
    v_n = kvn[:, C:].reshape(K, T, C)

    k_all = jnp.concatenate([k_s[None], k_n], axis=0)
    v_all = jnp.concatenate([v_s[None], v_n], axis=0)

    gi = jax.lax.broadcasted_iota(jnp.int32, (C, H), 0) // dh
    gh = jax.lax.broadcasted_iota(jnp.int32, (C, H), 1)
    G = (gi == gh).astype(jnp.float32)
    gti = jax.lax.broadcasted_iota(jnp.int32, (H, C), 1) // dh
    gth = jax.lax.broadcasted_iota(jnp.int32, (H, C), 0)
    Gt = (gti == gth).astype(jnp.float32)

    prod = q0[None] * k_all
    s = jax.lax.dot_general(prod.reshape((K + 1) * T, C), G,
                            (((1,), (0,)), ((), ())),
                            preferred_element_type=jnp.float32)
    s = s.reshape(K + 1, T, H) * jnp.float32(1.0 / np.sqrt(dh))

    vm = vm_ref[0]
    mask9 = jnp.concatenate([jnp.ones((1, T), jnp.float32), vm], axis=0)
    s = jnp.where(mask9[:, :, None] > 0.0, s, jnp.float32(-1e30))
    mx = jnp.max(s, axis=0, keepdims=True)
    e = jnp.exp(s - mx)
    w = e / jnp.sum(e, axis=0, keepdims=True)

    wb = jax.lax.dot_general(w.reshape((K + 1) * T, H), Gt,
                             (((1,), (0,)), ((), ())),
                             preferred_element_type=jnp.float32)
    o = jnp.sum(wb.reshape(K + 1, T, C) * v_all, axis=0)

    attn = jax.lax.dot_general(o, woT_ref[...], (((1,), (0,)), ((), ())),
                               preferred_element_type=jnp.float32) + bo_ref[...]

    x = x0 + attn
    mu = jnp.mean(x, axis=-1, keepdims=True)
    xc = x - mu
    var = jnp.mean(xc * xc, axis=-1, keepdims=True)
    xn = xc / jnp.sqrt(var + 1e-5) * g1_ref[...] + be1_ref[...]

    h1 = jax.lax.dot_general(xn, w1T_ref[...], (((1,), (0,)), ((), ())),
                             preferred_element_type=jnp.float32) + b1_ref[...]
    h1 = jnp.maximum(h1, 0.0)
    f = jax.lax.dot_general(h1, w2T_ref[...], (((1,), (0,)), ((), ())),
                            preferred_element_type=jnp.float32) + b2_ref[...]

    x2 = xn + f
    mu2 = jnp.mean(x2, axis=-1, keepdims=True)
    xc2 = x2 - mu2
    var2 = jnp.mean(xc2 * xc2, axis=-1, keepdims=True)
    y = xc2 / jnp.sqrt(var2 + 1e-5) * g2_ref[...] + be2_ref[...]

    out_ref[0] = y + x0


def _fuse_call(q_feat, neigh, vmask, wqkvT, bqkv, woT, bo, w1T, b1, w2T, b2,
               g1, be1, g2, be2):
    B, N, C = q_feat.shape
    FF = w1T.shape[-1]
    grid = (B, N // TB)
    full = lambda shape: pl.BlockSpec(shape, lambda b, t: tuple(0 for _ in shape))
    return pl.pallas_call(
        _fuse_body,
        grid=grid,
        in_specs=[
            pl.BlockSpec((1, TB, C), lambda b, t: (b, t, 0)),
            pl.BlockSpec((1, K, TB, C), lambda b, t: (b, 0, t, 0)),
            pl.BlockSpec((1, K, TB), lambda b, t: (b, 0, t)),
            full((C, 3 * C)),
            full((1, 3 * C)),
            full((C, C)),
            full((1, C)),
            full((C, FF)),
            full((1, FF)),
            full((FF, C)),
            full((1, C)),
            full((1, C)),
            full((1, C)),
            full((1, C)),
            full((1, C)),
        ],
        out_specs=pl.BlockSpec((1, TB, C), lambda b, t: (b, t, 0)),
        out_shape=jax.ShapeDtypeStruct((B, N, C), jnp.float32),
    )(q_feat, neigh, vmask, wqkvT, bqkv, woT, bo, w1T, b1, w2T, b2,
      g1, be1, g2, be2)


def kernel(q_xyz, q_feat, kv_xyz, kv_feat, Wqkv, bqkv, Wo, bo, W1, b1, W2, b2,
           g1, be1, g2, be2):
    B, N, C = q_feat.shape
    M = kv_xyz.shape[1]

    qpad = jnp.concatenate([q_xyz, jnp.zeros((B, N, 5), jnp.float32)], axis=-1)
    kpad = jnp.concatenate([kv_xyz, jnp.zeros((B, M, 5), jnp.float32)], axis=-1)
    kpadT = jnp.transpose(kpad, (0, 2, 1))

    idxT, vmaskT = _topk_call(qpad, kpadT)

    flat_idx = (idxT + (jnp.arange(B, dtype=jnp.int32) * M)[:, None, None])
    idx2d = flat_idx.reshape(B * K * N // 128, 128)
    table = kv_feat.reshape(B * M, C)

    neigh_flat = _sc_gather_call(table, idx2d)
    neigh = neigh_flat.reshape(B, K, N, C)

    return _fuse_call(
        q_feat, neigh, vmaskT,
        jnp.transpose(Wqkv), bqkv.reshape(1, 3 * C),
        jnp.transpose(Wo), bo.reshape(1, C),
        jnp.transpose(W1), b1.reshape(1, W1.shape[0]),
        jnp.transpose(W2), b2.reshape(1, C),
        g1.reshape(1, C), be1.reshape(1, C),
        g2.reshape(1, C), be2.reshape(1, C),
    )
